# Initial kernel scaffold; baseline (speedup 1.0000x reference)
import jax
import jax.numpy as jnp
from jax.experimental import pallas as pl


def kernel(heatmap):
    raise NotImplementedError("write your pallas kernel here")


# speedup vs baseline: 1.2999x; 1.2999x over previous
import functools

import jax
import jax.numpy as jnp
from jax.experimental import pallas as pl
from jax.experimental.pallas import tpu as pltpu

K_MAX = 20
NMS_THRESHOLD = 0.3
NEG_INF = float("-inf")


def _topk_body(x_ref, vals_ref, cx_ref, cy_ref, peaks_ref):
    H, W = peaks_ref.shape
    x = x_ref[0, 0]

    minf_row = jnp.full((1, W), NEG_INF, dtype=jnp.float32)
    up = jnp.concatenate([x[1:, :], minf_row], axis=0)
    down = jnp.concatenate([minf_row, x[:-1, :]], axis=0)
    vert = jnp.maximum(jnp.maximum(up, down), x)

    minf_col = jnp.full((H, 1), NEG_INF, dtype=jnp.float32)
    left = jnp.concatenate([vert[:, 1:], minf_col], axis=1)
    right = jnp.concatenate([minf_col, vert[:, :-1]], axis=1)
    pooled = jnp.maximum(jnp.maximum(left, right), vert)

    peaks = x * (pooled == x).astype(jnp.float32)
    peaks_ref[:, :] = peaks

    rowmax = jnp.max(peaks, axis=1).reshape(1, H)

    lane_iota = jax.lax.broadcasted_iota(jnp.int32, (1, W), 1)
    kvec_iota = jax.lax.broadcasted_iota(jnp.int32, (1, 128), 1)

    def body(k, carry):
        rowmax, vals, rows, cols = carry
        m = jnp.max(rowmax)
        r = jnp.min(jnp.where(rowmax == m, lane_iota, H))
        row = peaks_ref[pl.ds(r, 1), :]
        c = jnp.min(jnp.where(row == m, lane_iota, W))
        sel = kvec_iota == k
        vals = jnp.where(sel, m, vals)
        rows = jnp.where(sel, r.astype(jnp.float32), rows)
        cols = jnp.where(sel, c.astype(jnp.float32), cols)
        row_upd = jnp.where(lane_iota == c, NEG_INF, row)
        peaks_ref[pl.ds(r, 1), :] = row_upd
        rowmax = jnp.where(lane_iota == r, jnp.max(row_upd), rowmax)
        return rowmax, vals, rows, cols

    zero = jnp.zeros((1, 128), dtype=jnp.float32)
    _, vals, rows, cols = jax.lax.fori_loop(
        0, K_MAX, body, (rowmax, zero, zero, zero))

    validf = (vals >= NMS_THRESHOLD).astype(jnp.float32)
    cx = (2.0 * cols / jnp.float32(W - 1) - 1.0) * validf
    cy = (2.0 * rows / jnp.float32(H - 1) - 1.0) * validf
    vals_ref[0] = vals
    cx_ref[0] = cx
    cy_ref[0] = cy


@jax.jit
def kernel(heatmap):
    B, _, H, W = heatmap.shape
    out_shape = jax.ShapeDtypeStruct((B, 1, 128), jnp.float32)
    vals, cx, cy = pl.pallas_call(
        _topk_body,
        grid=(B,),
        in_specs=[pl.BlockSpec((1, 1, H, W), lambda b: (b, 0, 0, 0))],
        out_specs=[
            pl.BlockSpec((1, 1, 128), lambda b: (b, 0, 0)),
            pl.BlockSpec((1, 1, 128), lambda b: (b, 0, 0)),
            pl.BlockSpec((1, 1, 128), lambda b: (b, 0, 0)),
        ],
        out_shape=[out_shape, out_shape, out_shape],
        scratch_shapes=[pltpu.VMEM((H, W), jnp.float32)],
    )(heatmap)
    top_vals = vals[:, 0, :K_MAX]
    centers = jnp.stack([cx[:, 0, :K_MAX], cy[:, 0, :K_MAX]], axis=-1)
    valid_mask = top_vals >= NMS_THRESHOLD
    return centers, valid_mask, top_vals
